# Initial kernel scaffold; baseline (speedup 1.0000x reference)
import jax
import jax.numpy as jnp
from jax.experimental import pallas as pl


def kernel(x, ed_idx, adj, W, b):
    raise NotImplementedError("write your pallas kernel here")


# speedup vs baseline: 1.3093x; 1.3093x over previous
import functools
import jax
import jax.numpy as jnp
from jax.experimental import pallas as pl
from jax.experimental.pallas import tpu as pltpu

N = 8192
DIM = 128
NC = 8
B = 8
ROWS = 128
STEPS = N // ROWS


def _tc_kernel(ed_ref, x_ref, W_ref, b_ref, adj_ref, out_ref, acc_ref, hid_ref):
    i = pl.program_id(0)

    @pl.when(i == 0)
    def _init():
        acc_ref[...] = jnp.zeros_like(acc_ref)
        hid_ref[...] = (
            jnp.dot(x_ref[...], W_ref[...], preferred_element_type=jnp.float32)
            + b_ref[...]
        )

    base = i * ROWS
    s0 = jnp.int32(0)
    s1 = jnp.int32(0)
    for k in range(B):
        e = ed_ref[k]
        s0 = s0 + jnp.where(e <= base, 1, 0).astype(jnp.int32)
        s1 = s1 + jnp.where(e <= base + ROWS - 1, 1, 0).astype(jnp.int32)

    @pl.when(s0 == s1)
    def _fast():
        col = jnp.sum(adj_ref[...], axis=0, keepdims=True)
        acc_ref[pl.ds(s0, 1), :] += col

    @pl.when(s0 != s1)
    def _slow():
        rid = base + jax.lax.broadcasted_iota(jnp.int32, (ROWS, 1), 0)
        cnt = jnp.zeros((ROWS, 1), jnp.int32)
        for k in range(B):
            cnt = cnt + jnp.where(ed_ref[k] <= rid, 1, 0).astype(jnp.int32)
        tile = adj_ref[...]
        for s in range(B + 1):
            mask = (cnt == s).astype(jnp.float32)
            acc_ref[pl.ds(s, 1), :] += jnp.sum(tile * mask, axis=0, keepdims=True)

    @pl.when(i == STEPS - 1)
    def _final():
        out_ref[...] = jnp.dot(
            acc_ref[0:B, :], hid_ref[...], preferred_element_type=jnp.float32
        )


def kernel(x, ed_idx, adj, W, b):
    ed16 = jnp.concatenate([ed_idx, jnp.full((8,), N, jnp.int32)])
    b2 = b.reshape(1, NC)
    grid_spec = pltpu.PrefetchScalarGridSpec(
        num_scalar_prefetch=1,
        grid=(STEPS,),
        in_specs=[
            pl.BlockSpec((N, DIM), lambda i, *_: (0, 0)),
            pl.BlockSpec((DIM, NC), lambda i, *_: (0, 0)),
            pl.BlockSpec((1, NC), lambda i, *_: (0, 0)),
            pl.BlockSpec((ROWS, N), lambda i, *_: (i, 0)),
        ],
        out_specs=pl.BlockSpec((B, NC), lambda i, *_: (0, 0)),
        scratch_shapes=[
            pltpu.VMEM((16, N), jnp.float32),
            pltpu.VMEM((N, NC), jnp.float32),
        ],
    )
    return pl.pallas_call(
        _tc_kernel,
        grid_spec=grid_spec,
        out_shape=jax.ShapeDtypeStruct((B, NC), jnp.float32),
    )(ed16, x, W, b2, adj)
